# baseline (speedup 1.0000x reference)
import functools

import jax
import jax.numpy as jnp
from jax import lax
from jax.experimental import pallas as pl
from jax.experimental.pallas import tpu as pltpu
from jax.experimental.pallas import tpu_sc as plsc

TOPIC_K = 128
DOC_NUM = 100000
B = 1024
N1 = 32
N2 = 16
V = 8192

NW = 32
BPW = B // NW
X_AT = 0
TWO_AT = BPW
ONE_AT = TWO_AT + N2
ROWS = 64
F1_BASE = B
M2_BASE = B + N1


def _sc_gather(comb, table):
    info = plsc.get_sparse_core_info()
    nc = info.num_cores
    mesh = plsc.VectorSubcoreMesh(core_axis_name="c", subcore_axis_name="s")

    @functools.partial(
        pl.kernel,
        mesh=mesh,
        out_type=jax.ShapeDtypeStruct((B + 2 * N1, TOPIC_K), jnp.float32),
        scratch_types=[
            pltpu.VMEM((ROWS,), jnp.int32),
            pltpu.VMEM((ROWS, TOPIC_K), jnp.float32),
            pltpu.VMEM((1, TOPIC_K), jnp.float32),
            pltpu.SemaphoreType.DMA,
            pltpu.SemaphoreType.DMA,
        ],
    )
    def k(comb_hbm, table_hbm, g_out, idx_v, rows_v, mean_v, sem, sem2):
        wid = lax.axis_index("s") * nc + lax.axis_index("c")
        pltpu.sync_copy(comb_hbm.at[wid], idx_v)
        pltpu.async_copy(table_hbm.at[idx_v], rows_v, sem).wait()
        xcp = pltpu.async_copy(rows_v.at[pl.ds(X_AT, BPW)],
                               g_out.at[pl.ds(wid * BPW, BPW)], sem2)
        fcp = pltpu.async_copy(rows_v.at[pl.ds(ONE_AT, 1)],
                               g_out.at[pl.ds(F1_BASE + wid, 1)], sem2)
        inv = jnp.float32(1.0 / N2)

        def mean_step(j, accs):
            return tuple(a + rows_v[TWO_AT + j, pl.ds(c * 16, 16)]
                         for c, a in enumerate(accs))

        accs = tuple(rows_v[TWO_AT, pl.ds(c * 16, 16)]
                     for c in range(TOPIC_K // 16))
        accs = lax.fori_loop(1, N2, mean_step, accs)
        for c in range(TOPIC_K // 16):
            mean_v[0, pl.ds(c * 16, 16)] = accs[c] * inv
        pltpu.sync_copy(mean_v, g_out.at[pl.ds(M2_BASE + wid, 1)])
        xcp.wait()
        fcp.wait()

    return k(comb, table)


VCHUNK = 2048
NCHUNK = V // VCHUNK


def _w_body(wm_ref, aww_ref, w_ref):
    w_ref[...] = lax.dot_general(wm_ref[...], aww_ref[...],
                                 (((1,), (1,)), ((), ())))


def _precompute_w(W_mat, att_W_w):
    return pl.pallas_call(
        _w_body,
        grid=(NCHUNK,),
        in_specs=[
            pl.BlockSpec((VCHUNK, TOPIC_K), lambda i: (i, 0)),
            pl.BlockSpec((TOPIC_K, TOPIC_K), lambda i: (0, 0)),
        ],
        out_specs=pl.BlockSpec((VCHUNK, TOPIC_K), lambda i: (i, 0)),
        out_shape=jax.ShapeDtypeStruct((V, TOPIC_K), jnp.float32),
    )(W_mat, att_W_w)


def _tc_body(g_ref, wm_ref, w_ref, c2w_ref, c2b_ref, s2w_ref,
             c1w_ref, c1b_ref, s1w_ref, adw_ref, out_ref,
             h_s, d_s, logits_s):
    dot_t = lambda a, b: lax.dot_general(a, b, (((1,), (1,)), ((), ())))
    dot = lambda a, b: lax.dot_general(a, b, (((1,), (0,)), ((), ())))

    def norm_rows(t):
        n = jnp.sqrt(jnp.sum(t * t, axis=-1, keepdims=True))
        return t / jnp.maximum(n, 1e-12)

    i = pl.program_id(0)

    @pl.when(i == 0)
    def _prologue():
        f1 = g_ref[pl.ds(F1_BASE, N1), :]
        m2 = g_ref[pl.ds(M2_BASE, N1), :]
        o = (dot_t(m2, c2w_ref[...]) + c2b_ref[...]
             + dot_t(f1, s2w_ref[...]))
        o = norm_rows(jnp.maximum(o, 0.0))
        nm1 = jnp.mean(o, axis=0, keepdims=True)
        t1 = dot_t(nm1, c1w_ref[...]) + c1b_ref[...]
        x = g_ref[pl.ds(0, B), :]
        h = jnp.maximum(dot_t(x, s1w_ref[...]) + t1, 0.0)
        h = norm_rows(h)
        h_s[...] = h
        d_s[...] = dot_t(h, adw_ref[...])
        logits_s[...] = jnp.zeros((B, TOPIC_K), jnp.float32)

    wm_c = wm_ref[...]
    logits_s[...] += dot(dot_t(d_s[...], w_ref[...]), wm_c)

    @pl.when(i == NCHUNK - 1)
    def _epilogue():
        logits = logits_s[...]
        m = jnp.max(logits, axis=-1, keepdims=True)
        p = jnp.exp(logits - m)
        att = p / jnp.sum(p, axis=-1, keepdims=True)
        out_ref[...] = norm_rows(h_s[...] * att)


def kernel(v, one_hop_list, two_hop_list, W_mat, topic_dist, conv2_w, conv2_b,
           self2_w, conv1_w, conv1_b, self1_w, att_d_w, att_W_w):
    xs = v.astype(jnp.int32).reshape(NW, BPW)
    one = one_hop_list.astype(jnp.int32).reshape(N1, 1)
    comb = jnp.concatenate(
        [xs, two_hop_list.astype(jnp.int32),
         jnp.broadcast_to(one, (N1, ROWS - ONE_AT))], axis=1)
    g = _sc_gather(comb, topic_dist)
    w_full = _precompute_w(W_mat, att_W_w)

    full = lambda s: pl.BlockSpec(s, lambda i: (0, 0))
    out = pl.pallas_call(
        _tc_body,
        grid=(NCHUNK,),
        in_specs=[
            full((B + 2 * N1, TOPIC_K)),
            pl.BlockSpec((VCHUNK, TOPIC_K), lambda i: (i, 0)),
            pl.BlockSpec((VCHUNK, TOPIC_K), lambda i: (i, 0)),
            full((TOPIC_K, TOPIC_K)),
            full((1, TOPIC_K)),
            full((TOPIC_K, TOPIC_K)),
            full((TOPIC_K, TOPIC_K)),
            full((1, TOPIC_K)),
            full((TOPIC_K, TOPIC_K)),
            full((TOPIC_K, TOPIC_K)),
        ],
        out_specs=full((B, TOPIC_K)),
        out_shape=jax.ShapeDtypeStruct((B, TOPIC_K), jnp.float32),
        scratch_shapes=[
            pltpu.VMEM((B, TOPIC_K), jnp.float32),
            pltpu.VMEM((B, TOPIC_K), jnp.float32),
            pltpu.VMEM((B, TOPIC_K), jnp.float32),
        ],
    )(g, W_mat, w_full, conv2_w, conv2_b.reshape(1, TOPIC_K),
      self2_w, conv1_w, conv1_b.reshape(1, TOPIC_K), self1_w, att_d_w)
    return out


# speedup vs baseline: 1.0229x; 1.0229x over previous
import functools

import jax
import jax.numpy as jnp
from jax import lax
from jax.experimental import pallas as pl
from jax.experimental.pallas import tpu as pltpu
from jax.experimental.pallas import tpu_sc as plsc

TOPIC_K = 128
DOC_NUM = 100000
B = 1024
N1 = 32
N2 = 16
V = 8192

NW = 32
BPW = B // NW
X_AT = 0
TWO_AT = BPW
ONE_AT = TWO_AT + N2
ROWS = 64
F1_BASE = B
M2_BASE = B + N1


def _sc_gather(comb, table):
    info = plsc.get_sparse_core_info()
    nc = info.num_cores
    mesh = plsc.VectorSubcoreMesh(core_axis_name="c", subcore_axis_name="s")

    @functools.partial(
        pl.kernel,
        mesh=mesh,
        out_type=jax.ShapeDtypeStruct((B + 2 * N1, TOPIC_K), jnp.float32),
        scratch_types=[
            pltpu.VMEM((ROWS,), jnp.int32),
            pltpu.VMEM((ROWS, TOPIC_K), jnp.float32),
            pltpu.VMEM((1, TOPIC_K), jnp.float32),
            pltpu.SemaphoreType.DMA,
            pltpu.SemaphoreType.DMA,
        ],
    )
    def k(comb_hbm, table_hbm, g_out, idx_v, rows_v, mean_v, sem, sem2):
        wid = lax.axis_index("s") * nc + lax.axis_index("c")
        pltpu.sync_copy(comb_hbm.at[wid], idx_v)
        pltpu.async_copy(table_hbm.at[idx_v], rows_v, sem).wait()
        xcp = pltpu.async_copy(rows_v.at[pl.ds(X_AT, BPW)],
                               g_out.at[pl.ds(wid * BPW, BPW)], sem2)
        fcp = pltpu.async_copy(rows_v.at[pl.ds(ONE_AT, 1)],
                               g_out.at[pl.ds(F1_BASE + wid, 1)], sem2)
        inv = jnp.float32(1.0 / N2)

        def mean_step(j, accs):
            return tuple(a + rows_v[TWO_AT + j, pl.ds(c * 16, 16)]
                         for c, a in enumerate(accs))

        accs = tuple(rows_v[TWO_AT, pl.ds(c * 16, 16)]
                     for c in range(TOPIC_K // 16))
        accs = lax.fori_loop(1, N2, mean_step, accs)
        for c in range(TOPIC_K // 16):
            mean_v[0, pl.ds(c * 16, 16)] = accs[c] * inv
        pltpu.sync_copy(mean_v, g_out.at[pl.ds(M2_BASE + wid, 1)])
        xcp.wait()
        fcp.wait()

    return k(comb, table)


VCHUNK = 1024
NCHUNK = V // VCHUNK


def _tc_body(g_ref, wm_ref, c2w_ref, c2b_ref, s2w_ref,
             c1w_ref, c1b_ref, s1w_ref, adw_ref, aww_ref, out_ref,
             h_s, d_s, logits_s):
    dot_t = lambda a, b: lax.dot_general(a, b, (((1,), (1,)), ((), ())))
    dot = lambda a, b: lax.dot_general(a, b, (((1,), (0,)), ((), ())))

    def norm_rows(t):
        n = jnp.sqrt(jnp.sum(t * t, axis=-1, keepdims=True))
        return t / jnp.maximum(n, 1e-12)

    i = pl.program_id(0)

    @pl.when(i == 0)
    def _prologue():
        f1 = g_ref[pl.ds(F1_BASE, N1), :]
        m2 = g_ref[pl.ds(M2_BASE, N1), :]
        o = (dot_t(m2, c2w_ref[...]) + c2b_ref[...]
             + dot_t(f1, s2w_ref[...]))
        o = norm_rows(jnp.maximum(o, 0.0))
        nm1 = jnp.mean(o, axis=0, keepdims=True)
        t1 = dot_t(nm1, c1w_ref[...]) + c1b_ref[...]
        x = g_ref[pl.ds(0, B), :]
        h = jnp.maximum(dot_t(x, s1w_ref[...]) + t1, 0.0)
        h = norm_rows(h)
        h_s[...] = h
        d_s[...] = dot_t(h, adw_ref[...])
        logits_s[...] = jnp.zeros((B, TOPIC_K), jnp.float32)

    wm_c = wm_ref[...]
    w_c = dot_t(wm_c, aww_ref[...])
    logits_s[...] += dot(dot_t(d_s[...], w_c), wm_c)

    @pl.when(i == NCHUNK - 1)
    def _epilogue():
        logits = logits_s[...]
        m = jnp.max(logits, axis=-1, keepdims=True)
        p = jnp.exp(logits - m)
        att = p / jnp.sum(p, axis=-1, keepdims=True)
        out_ref[...] = norm_rows(h_s[...] * att)


def kernel(v, one_hop_list, two_hop_list, W_mat, topic_dist, conv2_w, conv2_b,
           self2_w, conv1_w, conv1_b, self1_w, att_d_w, att_W_w):
    xs = v.astype(jnp.int32).reshape(NW, BPW)
    one = one_hop_list.astype(jnp.int32).reshape(N1, 1)
    comb = jnp.concatenate(
        [xs, two_hop_list.astype(jnp.int32),
         jnp.broadcast_to(one, (N1, ROWS - ONE_AT))], axis=1)
    g = _sc_gather(comb, topic_dist)

    full = lambda s: pl.BlockSpec(s, lambda i: (0, 0))
    out = pl.pallas_call(
        _tc_body,
        grid=(NCHUNK,),
        in_specs=[
            full((B + 2 * N1, TOPIC_K)),
            pl.BlockSpec((VCHUNK, TOPIC_K), lambda i: (i, 0)),
            full((TOPIC_K, TOPIC_K)),
            full((1, TOPIC_K)),
            full((TOPIC_K, TOPIC_K)),
            full((TOPIC_K, TOPIC_K)),
            full((1, TOPIC_K)),
            full((TOPIC_K, TOPIC_K)),
            full((TOPIC_K, TOPIC_K)),
            full((TOPIC_K, TOPIC_K)),
        ],
        out_specs=full((B, TOPIC_K)),
        out_shape=jax.ShapeDtypeStruct((B, TOPIC_K), jnp.float32),
        scratch_shapes=[
            pltpu.VMEM((B, TOPIC_K), jnp.float32),
            pltpu.VMEM((B, TOPIC_K), jnp.float32),
            pltpu.VMEM((B, TOPIC_K), jnp.float32),
        ],
    )(g, W_mat, conv2_w, conv2_b.reshape(1, TOPIC_K),
      self2_w, conv1_w, conv1_b.reshape(1, TOPIC_K), self1_w, att_d_w, att_W_w)
    return out


# speedup vs baseline: 1.0736x; 1.0495x over previous
import functools

import jax
import jax.numpy as jnp
from jax import lax
from jax.experimental import pallas as pl
from jax.experimental.pallas import tpu as pltpu
from jax.experimental.pallas import tpu_sc as plsc

TOPIC_K = 128
DOC_NUM = 100000
B = 1024
N1 = 32
N2 = 16
V = 8192

NW = 32
BPW = B // NW
X_AT = 0
TWO_AT = BPW
ONE_AT = TWO_AT + N2
ROWS = 64
F1_BASE = B
M2_BASE = B + N1


def _sc_gather(comb, table):
    info = plsc.get_sparse_core_info()
    nc = info.num_cores
    mesh = plsc.VectorSubcoreMesh(core_axis_name="c", subcore_axis_name="s")

    @functools.partial(
        pl.kernel,
        mesh=mesh,
        out_type=jax.ShapeDtypeStruct((B + 2 * N1, TOPIC_K), jnp.float32),
        scratch_types=[
            pltpu.VMEM((ROWS,), jnp.int32),
            pltpu.VMEM((ROWS, TOPIC_K), jnp.float32),
            pltpu.VMEM((1, TOPIC_K), jnp.float32),
            pltpu.SemaphoreType.DMA,
            pltpu.SemaphoreType.DMA,
        ],
    )
    def k(comb_hbm, table_hbm, g_out, idx_v, rows_v, mean_v, sem, sem2):
        wid = lax.axis_index("s") * nc + lax.axis_index("c")
        pltpu.sync_copy(comb_hbm.at[wid], idx_v)
        pltpu.async_copy(table_hbm.at[idx_v], rows_v, sem).wait()
        xcp = pltpu.async_copy(rows_v.at[pl.ds(X_AT, BPW)],
                               g_out.at[pl.ds(wid * BPW, BPW)], sem2)
        fcp = pltpu.async_copy(rows_v.at[pl.ds(ONE_AT, 1)],
                               g_out.at[pl.ds(F1_BASE + wid, 1)], sem2)
        inv = jnp.float32(1.0 / N2)

        def mean_step(j, accs):
            return tuple(a + rows_v[TWO_AT + j, pl.ds(c * 16, 16)]
                         for c, a in enumerate(accs))

        accs = tuple(rows_v[TWO_AT, pl.ds(c * 16, 16)]
                     for c in range(TOPIC_K // 16))
        accs = lax.fori_loop(1, N2, mean_step, accs)
        for c in range(TOPIC_K // 16):
            mean_v[0, pl.ds(c * 16, 16)] = accs[c] * inv
        pltpu.sync_copy(mean_v, g_out.at[pl.ds(M2_BASE + wid, 1)])
        xcp.wait()
        fcp.wait()

    return k(comb, table)


VCHUNK = 4096
NCHUNK = V // VCHUNK


def _tc_body(g_ref, wm_ref, c2w_ref, c2b_ref, s2w_ref,
             c1w_ref, c1b_ref, s1w_ref, adw_ref, aww_ref, out_ref,
             h_s, d_s, logits_s):
    dot_t = lambda a, b: lax.dot_general(a, b, (((1,), (1,)), ((), ())))
    dot = lambda a, b: lax.dot_general(a, b, (((1,), (0,)), ((), ())))

    def norm_rows(t):
        n = jnp.sqrt(jnp.sum(t * t, axis=-1, keepdims=True))
        return t / jnp.maximum(n, 1e-12)

    i = pl.program_id(0)

    @pl.when(i == 0)
    def _prologue():
        f1 = g_ref[pl.ds(F1_BASE, N1), :]
        m2 = g_ref[pl.ds(M2_BASE, N1), :]
        o = (dot_t(m2, c2w_ref[...]) + c2b_ref[...]
             + dot_t(f1, s2w_ref[...]))
        o = norm_rows(jnp.maximum(o, 0.0))
        nm1 = jnp.mean(o, axis=0, keepdims=True)
        t1 = dot_t(nm1, c1w_ref[...]) + c1b_ref[...]
        x = g_ref[pl.ds(0, B), :]
        h = jnp.maximum(dot_t(x, s1w_ref[...]) + t1, 0.0)
        h = norm_rows(h)
        h_s[...] = h
        d_s[...] = dot_t(h, adw_ref[...])
        logits_s[...] = jnp.zeros((B, TOPIC_K), jnp.float32)

    wm_c = wm_ref[...]
    w_c = dot_t(wm_c, aww_ref[...])
    logits_s[...] += dot(dot_t(d_s[...], w_c), wm_c)

    @pl.when(i == NCHUNK - 1)
    def _epilogue():
        logits = logits_s[...]
        m = jnp.max(logits, axis=-1, keepdims=True)
        p = jnp.exp(logits - m)
        att = p / jnp.sum(p, axis=-1, keepdims=True)
        out_ref[...] = norm_rows(h_s[...] * att)


def kernel(v, one_hop_list, two_hop_list, W_mat, topic_dist, conv2_w, conv2_b,
           self2_w, conv1_w, conv1_b, self1_w, att_d_w, att_W_w):
    xs = v.astype(jnp.int32).reshape(NW, BPW)
    one = one_hop_list.astype(jnp.int32).reshape(N1, 1)
    comb = jnp.concatenate(
        [xs, two_hop_list.astype(jnp.int32),
         jnp.broadcast_to(one, (N1, ROWS - ONE_AT))], axis=1)
    g = _sc_gather(comb, topic_dist)

    full = lambda s: pl.BlockSpec(s, lambda i: (0, 0))
    out = pl.pallas_call(
        _tc_body,
        grid=(NCHUNK,),
        in_specs=[
            full((B + 2 * N1, TOPIC_K)),
            pl.BlockSpec((VCHUNK, TOPIC_K), lambda i: (i, 0)),
            full((TOPIC_K, TOPIC_K)),
            full((1, TOPIC_K)),
            full((TOPIC_K, TOPIC_K)),
            full((TOPIC_K, TOPIC_K)),
            full((1, TOPIC_K)),
            full((TOPIC_K, TOPIC_K)),
            full((TOPIC_K, TOPIC_K)),
            full((TOPIC_K, TOPIC_K)),
        ],
        out_specs=full((B, TOPIC_K)),
        out_shape=jax.ShapeDtypeStruct((B, TOPIC_K), jnp.float32),
        scratch_shapes=[
            pltpu.VMEM((B, TOPIC_K), jnp.float32),
            pltpu.VMEM((B, TOPIC_K), jnp.float32),
            pltpu.VMEM((B, TOPIC_K), jnp.float32),
        ],
    )(g, W_mat, conv2_w, conv2_b.reshape(1, TOPIC_K),
      self2_w, conv1_w, conv1_b.reshape(1, TOPIC_K), self1_w, att_d_w, att_W_w)
    return out
